# Initial kernel scaffold; baseline (speedup 1.0000x reference)
import jax
import jax.numpy as jnp
from jax.experimental import pallas as pl


def kernel(x, Wr, gate, up, down):
    raise NotImplementedError("write your pallas kernel here")


# speedup vs baseline: 1.0524x; 1.0524x over previous
import jax
import jax.numpy as jnp
from jax.experimental import pallas as pl
from jax.experimental.pallas import tpu as pltpu

_B, _S, _D = 2, 4096, 1024
_E, _K, _FF = 8, 2, 4096

_TT = 1024
_FB = 512


def _moe_kernel(x_ref, w_ref, gate_ref, up_ref, down_ref, out_ref, h_scr):
    e = pl.program_id(1)
    f = pl.program_id(2)

    xt_bf = x_ref[...].astype(jnp.bfloat16)
    g = gate_ref[0]
    u = up_ref[0]
    dn = down_ref[0]
    a = jax.lax.dot_general(xt_bf, g,
                            dimension_numbers=(((1,), (1,)), ((), ())),
                            preferred_element_type=jnp.float32)
    b = jax.lax.dot_general(xt_bf, u,
                            dimension_numbers=(((1,), (1,)), ((), ())),
                            preferred_element_type=jnp.float32)
    h_scr[...] = (a * jax.lax.logistic(a) * b)
    h_bf = h_scr[...].astype(jnp.bfloat16)
    part = jax.lax.dot_general(h_bf, dn,
                               dimension_numbers=(((1,), (1,)), ((), ())),
                               preferred_element_type=jnp.float32)
    lane_e = jax.lax.broadcasted_iota(jnp.int32, (_TT, _E), 1)
    w_e = jnp.sum(jnp.where(lane_e == e, w_ref[...], 0.0),
                  axis=1, keepdims=True)
    contrib = w_e * part

    @pl.when(jnp.logical_and(e == 0, f == 0))
    def _init():
        out_ref[...] = contrib

    @pl.when(jnp.logical_not(jnp.logical_and(e == 0, f == 0)))
    def _acc():
        out_ref[...] += contrib


def kernel(x, Wr, gate, up, down):
    b, s, d = x.shape
    T = b * s
    xf = x.reshape(T, d)

    router_logits = xf @ Wr.T
    routing_weights = jax.nn.softmax(router_logits.astype(jnp.float32), axis=1)
    top_w, top_i = jax.lax.top_k(routing_weights, _K)
    w = jnp.sum(top_w[:, :, None]
                * (top_i[:, :, None] == jnp.arange(_E)[None, None, :]),
                axis=1).astype(jnp.float32)

    gate_bf = gate.astype(jnp.bfloat16)
    up_bf = up.astype(jnp.bfloat16)
    down_bf = down.astype(jnp.bfloat16)

    grid = (T // _TT, _E, _FF // _FB)

    out = pl.pallas_call(
        _moe_kernel,
        grid=grid,
        in_specs=[
            pl.BlockSpec((_TT, _D), lambda t, e, f: (t, 0)),
            pl.BlockSpec((_TT, _E), lambda t, e, f: (t, 0)),
            pl.BlockSpec((1, _FB, _D), lambda t, e, f: (e, f, 0)),
            pl.BlockSpec((1, _FB, _D), lambda t, e, f: (e, f, 0)),
            pl.BlockSpec((1, _D, _FB), lambda t, e, f: (e, 0, f)),
        ],
        out_specs=pl.BlockSpec((_TT, _D), lambda t, e, f: (t, 0)),
        out_shape=jax.ShapeDtypeStruct((T, _D), jnp.float32),
        scratch_shapes=[
            pltpu.VMEM((_TT, _FB), jnp.float32),
        ],
        compiler_params=pltpu.CompilerParams(
            dimension_semantics=("parallel", "arbitrary", "arbitrary"),
        ),
    )(xf, w, gate_bf, up_bf, down_bf)

    return out.reshape(b, s, d), router_logits


# speedup vs baseline: 1.8481x; 1.7561x over previous
import jax
import jax.numpy as jnp
from jax.experimental import pallas as pl
from jax.experimental.pallas import tpu as pltpu

_B, _S, _D = 2, 4096, 1024
_E, _K, _FF = 8, 2, 4096

_TS = 256
_N = _B * _S * _K
_CAP = _N + _E * _TS
_NT = _CAP // _TS


def _group_ffn_kernel(eot_ref, xs_ref, gate_ref, up_ref, down_ref, ys_ref):
    xt = xs_ref[...]
    g = gate_ref[0]
    u = up_ref[0]
    dn = down_ref[0]
    a = jax.lax.dot_general(xt, g,
                            dimension_numbers=(((1,), (1,)), ((), ())),
                            preferred_element_type=jnp.float32)
    b = jax.lax.dot_general(xt, u,
                            dimension_numbers=(((1,), (1,)), ((), ())),
                            preferred_element_type=jnp.float32)
    h = (a * jax.lax.logistic(a) * b).astype(jnp.bfloat16)
    ys_ref[...] = jax.lax.dot_general(
        h, dn,
        dimension_numbers=(((1,), (1,)), ((), ())),
        preferred_element_type=jnp.float32)


def kernel(x, Wr, gate, up, down):
    b, s, d = x.shape
    T = b * s
    xf = x.reshape(T, d)

    router_logits = xf @ Wr.T
    routing_weights = jax.nn.softmax(router_logits.astype(jnp.float32), axis=1)
    top_w, top_i = jax.lax.top_k(routing_weights, _K)

    expert_flat = top_i.reshape(-1).astype(jnp.int32)
    token_flat = (jnp.arange(_N, dtype=jnp.int32) // _K)
    order = jnp.argsort(expert_flat, stable=True)
    counts = jnp.bincount(expert_flat, length=_E)
    cum = jnp.concatenate([jnp.zeros(1, counts.dtype), jnp.cumsum(counts)])
    padded = ((counts + _TS - 1) // _TS) * _TS
    pstart = jnp.concatenate([jnp.zeros(1, padded.dtype), jnp.cumsum(padded)])
    e_sorted = expert_flat[order]
    rank = jnp.arange(_N) - cum[e_sorted]
    p_sorted = (pstart[e_sorted] + rank).astype(jnp.int32)
    row_token = jnp.zeros(_CAP, jnp.int32).at[p_sorted].set(token_flat[order])
    pos = jnp.zeros(_N, jnp.int32).at[order].set(p_sorted).reshape(T, _K)
    expert_of_tile = jnp.clip(
        jnp.searchsorted(pstart[1:], jnp.arange(_NT) * _TS, side="right"),
        0, _E - 1).astype(jnp.int32)

    xbf = xf.astype(jnp.bfloat16)
    xs = xbf[row_token]

    gate_bf = gate.astype(jnp.bfloat16)
    up_bf = up.astype(jnp.bfloat16)
    down_bf = down.astype(jnp.bfloat16)

    ys = pl.pallas_call(
        _group_ffn_kernel,
        grid_spec=pltpu.PrefetchScalarGridSpec(
            num_scalar_prefetch=1,
            grid=(_NT,),
            in_specs=[
                pl.BlockSpec((_TS, _D), lambda j, eot: (j, 0)),
                pl.BlockSpec((1, _FF, _D), lambda j, eot: (eot[j], 0, 0)),
                pl.BlockSpec((1, _FF, _D), lambda j, eot: (eot[j], 0, 0)),
                pl.BlockSpec((1, _D, _FF), lambda j, eot: (eot[j], 0, 0)),
            ],
            out_specs=pl.BlockSpec((_TS, _D), lambda j, eot: (j, 0)),
        ),
        out_shape=jax.ShapeDtypeStruct((_CAP, _D), jnp.float32),
        compiler_params=pltpu.CompilerParams(
            dimension_semantics=("arbitrary",),
        ),
    )(expert_of_tile, xs, gate_bf, up_bf, down_bf)

    ys_g = ys[pos]
    final = jnp.sum(top_w[:, :, None] * ys_g, axis=1)

    return final.reshape(b, s, d), router_logits
